# initial kernel scaffold (unmeasured)
import jax
import jax.numpy as jnp
from jax import lax
from jax.experimental import pallas as pl
from jax.experimental.pallas import tpu as pltpu


def kernel(
    x,
):
    def body(*refs):
        pass

    out_shape = jax.ShapeDtypeStruct(..., jnp.float32)
    return pl.pallas_call(body, out_shape=out_shape)(...)


# baseline (device time: 152170 ns/iter reference)
import jax
import jax.numpy as jnp
from jax import lax
from jax.experimental import pallas as pl
from jax.experimental.pallas import tpu as pltpu

N_DEV = 4


def kernel(x):
    m_per, n = x.shape

    def body(x_ref, out_ref, comm_ref, send_sems, recv_sems):
        my_pos = lax.axis_index("i")
        left = (my_pos - 1) % N_DEV
        right = (my_pos + 1) % N_DEV

        barrier_sem = pltpu.get_barrier_semaphore()
        for nbr in [left, right]:
            pl.semaphore_signal(
                barrier_sem, inc=1,
                device_id=(nbr,), device_id_type=pl.DeviceIdType.MESH,
            )
        pl.semaphore_wait(barrier_sem, 2)

        out_ref[pl.ds(my_pos * m_per, m_per), :] = x_ref[:, :]
        comm_ref[0, :, :] = x_ref[:, :]

        for h in range(N_DEV - 1):
            send_slot = h % 2
            recv_slot = (h + 1) % 2
            rdma = pltpu.make_async_remote_copy(
                src_ref=comm_ref.at[send_slot],
                dst_ref=comm_ref.at[recv_slot],
                send_sem=send_sems.at[send_slot],
                recv_sem=recv_sems.at[recv_slot],
                device_id=(right,),
                device_id_type=pl.DeviceIdType.MESH,
            )
            rdma.start()
            rdma.wait()

            origin = (my_pos - h - 1) % N_DEV
            out_ref[pl.ds(origin * m_per, m_per), :] = comm_ref[recv_slot, :, :]

    return pl.pallas_call(
        body,
        out_shape=jax.ShapeDtypeStruct((N_DEV * m_per, n), x.dtype),
        in_specs=[pl.BlockSpec(memory_space=pltpu.VMEM)],
        out_specs=pl.BlockSpec(memory_space=pltpu.VMEM),
        scratch_shapes=[
            pltpu.VMEM((2, m_per, n), x.dtype),
            pltpu.SemaphoreType.DMA((2,)),
            pltpu.SemaphoreType.DMA((2,)),
        ],
        compiler_params=pltpu.CompilerParams(collective_id=0),
    )(x)


# device time: 83733 ns/iter; 1.8173x vs baseline; 1.8173x over previous
import jax
import jax.numpy as jnp
from jax import lax
from jax.experimental import pallas as pl
from jax.experimental.pallas import tpu as pltpu

N_DEV = 4


def kernel(x):
    m_per, n = x.shape
    half = m_per // 2

    def body(x_ref, out_ref, send_sems, recv_sems):
        my_pos = lax.axis_index("i")
        left = (my_pos - 1) % N_DEV
        right = (my_pos + 1) % N_DEV

        barrier_sem = pltpu.get_barrier_semaphore()
        for nbr in [left, right]:
            pl.semaphore_signal(
                barrier_sem, inc=1,
                device_id=(nbr,), device_id_type=pl.DeviceIdType.MESH,
            )
        pl.semaphore_wait(barrier_sem, 2)

        out_ref[pl.ds(my_pos * m_per, m_per), :] = x_ref[:, :]

        rdmas = []
        for h in range(N_DEV - 1):
            for d, (dst, off) in enumerate([(right, 0), (left, half)]):
                origin = (my_pos - h) % N_DEV if d == 0 else (my_pos + h) % N_DEV
                row = origin * m_per + off
                rdma = pltpu.make_async_remote_copy(
                    src_ref=out_ref.at[pl.ds(row, half), :],
                    dst_ref=out_ref.at[pl.ds(row, half), :],
                    send_sem=send_sems.at[d, h],
                    recv_sem=recv_sems.at[d, h],
                    device_id=(dst,),
                    device_id_type=pl.DeviceIdType.MESH,
                )
                rdma.start()
                rdmas.append(rdma)
            if h < N_DEV - 2:
                rdmas[-2].wait_recv()
                rdmas[-1].wait_recv()

        rdmas[-2].wait_recv()
        rdmas[-1].wait_recv()
        for rdma in rdmas:
            rdma.wait_send()

    return pl.pallas_call(
        body,
        out_shape=jax.ShapeDtypeStruct((N_DEV * m_per, n), x.dtype),
        in_specs=[pl.BlockSpec(memory_space=pltpu.VMEM)],
        out_specs=pl.BlockSpec(memory_space=pltpu.VMEM),
        scratch_shapes=[
            pltpu.SemaphoreType.DMA((2, N_DEV - 1)),
            pltpu.SemaphoreType.DMA((2, N_DEV - 1)),
        ],
        compiler_params=pltpu.CompilerParams(collective_id=0),
    )(x)


# device time: 80665 ns/iter; 1.8864x vs baseline; 1.0380x over previous
import jax
import jax.numpy as jnp
from jax import lax
from jax.experimental import pallas as pl
from jax.experimental.pallas import tpu as pltpu

N_DEV = 4
SUB = 2


def kernel(x):
    m_per, n = x.shape
    half = m_per // 2
    sub = half // SUB

    def body(x_ref, out_ref, send_sems, recv_sems):
        my_pos = lax.axis_index("i")
        left = (my_pos - 1) % N_DEV
        right = (my_pos + 1) % N_DEV

        barrier_sem = pltpu.get_barrier_semaphore()
        for nbr in [left, right]:
            pl.semaphore_signal(
                barrier_sem, inc=1,
                device_id=(nbr,), device_id_type=pl.DeviceIdType.MESH,
            )
        pl.semaphore_wait(barrier_sem, 2)

        def start_hop(h, d, s, prev):
            dst = right if d == 0 else left
            origin = (my_pos - h) % N_DEV if d == 0 else (my_pos + h) % N_DEV
            off = (half if d == 1 else 0) + s * sub
            if h == 0:
                src = x_ref.at[pl.ds(off, sub), :]
            else:
                src = out_ref.at[pl.ds(origin * m_per + off, sub), :]
            rdma = pltpu.make_async_remote_copy(
                src_ref=src,
                dst_ref=out_ref.at[pl.ds(origin * m_per + off, sub), :],
                send_sem=send_sems.at[d, h, s],
                recv_sem=recv_sems.at[d, h, s],
                device_id=(dst,),
                device_id_type=pl.DeviceIdType.MESH,
            )
            if prev is not None:
                prev.wait_recv()
            rdma.start()
            return rdma

        rdmas = {}
        for s in range(SUB):
            for d in range(2):
                rdmas[0, d, s] = start_hop(0, d, s, None)

        out_ref[pl.ds(my_pos * m_per, m_per), :] = x_ref[:, :]

        for h in range(1, N_DEV - 1):
            for s in range(SUB):
                for d in range(2):
                    rdmas[h, d, s] = start_hop(h, d, s, rdmas[h - 1, d, s])

        for s in range(SUB):
            for d in range(2):
                rdmas[N_DEV - 2, d, s].wait_recv()
        for rdma in rdmas.values():
            rdma.wait_send()

    return pl.pallas_call(
        body,
        out_shape=jax.ShapeDtypeStruct((N_DEV * m_per, n), x.dtype),
        in_specs=[pl.BlockSpec(memory_space=pltpu.VMEM)],
        out_specs=pl.BlockSpec(memory_space=pltpu.VMEM),
        scratch_shapes=[
            pltpu.SemaphoreType.DMA((2, N_DEV - 1, SUB)),
            pltpu.SemaphoreType.DMA((2, N_DEV - 1, SUB)),
        ],
        compiler_params=pltpu.CompilerParams(collective_id=0),
    )(x)


# device time: 79400 ns/iter; 1.9165x vs baseline; 1.0159x over previous
import jax
import jax.numpy as jnp
from jax import lax
from jax.experimental import pallas as pl
from jax.experimental.pallas import tpu as pltpu

N_DEV = 4
SUB = 4


def kernel(x):
    m_per, n = x.shape
    half = m_per // 2
    sub = half // SUB

    def body(x_ref, out_ref, send_sems, recv_sems, copy_sem):
        my_pos = lax.axis_index("i")
        left = (my_pos - 1) % N_DEV
        right = (my_pos + 1) % N_DEV

        barrier_sem = pltpu.get_barrier_semaphore()
        for nbr in [left, right]:
            pl.semaphore_signal(
                barrier_sem, inc=1,
                device_id=(nbr,), device_id_type=pl.DeviceIdType.MESH,
            )
        pl.semaphore_wait(barrier_sem, 2)

        def start_hop(h, d, s, prev):
            dst = right if d == 0 else left
            origin = (my_pos - h) % N_DEV if d == 0 else (my_pos + h) % N_DEV
            off = (half if d == 1 else 0) + s * sub
            if h == 0:
                src = x_ref.at[pl.ds(off, sub), :]
            else:
                src = out_ref.at[pl.ds(origin * m_per + off, sub), :]
            rdma = pltpu.make_async_remote_copy(
                src_ref=src,
                dst_ref=out_ref.at[pl.ds(origin * m_per + off, sub), :],
                send_sem=send_sems.at[d, h, s],
                recv_sem=recv_sems.at[d, h, s],
                device_id=(dst,),
                device_id_type=pl.DeviceIdType.MESH,
            )
            if prev is not None:
                prev.wait_recv()
            rdma.start()
            return rdma

        rdmas = {}
        for s in range(SUB):
            for d in range(2):
                rdmas[0, d, s] = start_hop(0, d, s, None)

        local_copy = pltpu.make_async_copy(
            x_ref, out_ref.at[pl.ds(my_pos * m_per, m_per), :], copy_sem
        )
        local_copy.start()

        for h in range(1, N_DEV - 1):
            for s in range(SUB):
                for d in range(2):
                    rdmas[h, d, s] = start_hop(h, d, s, rdmas[h - 1, d, s])

        for s in range(SUB):
            for d in range(2):
                rdmas[N_DEV - 2, d, s].wait_recv()
        for rdma in rdmas.values():
            rdma.wait_send()
        local_copy.wait()

    return pl.pallas_call(
        body,
        out_shape=jax.ShapeDtypeStruct((N_DEV * m_per, n), x.dtype),
        in_specs=[pl.BlockSpec(memory_space=pltpu.VMEM)],
        out_specs=pl.BlockSpec(memory_space=pltpu.VMEM),
        scratch_shapes=[
            pltpu.SemaphoreType.DMA((2, N_DEV - 1, SUB)),
            pltpu.SemaphoreType.DMA((2, N_DEV - 1, SUB)),
            pltpu.SemaphoreType.DMA,
        ],
        compiler_params=pltpu.CompilerParams(collective_id=0),
    )(x)
